# baseline (speedup 1.0000x reference)
import jax
import jax.numpy as jnp
from jax import lax
from jax.experimental import pallas as pl
from jax.experimental.pallas import tpu as pltpu
from jax.experimental.pallas import tpu_sc as plsc

VOCAB = 1000000
DIM = 64
SEQ = 200
BATCH = 4096

NC, NS = 2, 16
NW = NC * NS
BPW = BATCH // NW
NB = 4
NT = 2


def _sc_kernel(xt_hbm, tab_hbm, pos_hbm, out_hbm, pos_v, idx_v, rows_v,
               tile_v, gsem, osem):
    wid = lax.axis_index("s") * NC + lax.axis_index("c")

    pltpu.sync_copy(pos_hbm, pos_v)
    pltpu.sync_copy(xt_hbm.at[:, pl.ds(wid * BPW, BPW)], idx_v)

    iota16 = lax.iota(jnp.int32, 16)
    db_idx = [iota16 // 8 + 2 * dg for dg in range(4)]
    di_idx = [lax.rem(iota16, 8)] * 4

    def gather(s, b):
        pltpu.async_copy(tab_hbm.at[idx_v.at[s]], rows_v.at[b], gsem.at[b])

    gather(0, 0)
    gather(1, 1)
    gather(2, 2)

    @pl.loop(0, SEQ)
    def s_loop(s):
        b = lax.rem(s, NB)
        tb = lax.rem(s, NT)

        @pl.when(s + 3 < SEQ)
        def _():
            gather(s + 3, lax.rem(s + 3, NB))

        pltpu.make_async_copy(tab_hbm.at[idx_v.at[s]], rows_v.at[b],
                              gsem.at[b]).wait()

        @pl.when(s >= NT)
        def _():
            pltpu.make_async_copy(tile_v.at[tb, :, :, pl.ds(0, BPW)],
                                  out_hbm.at[s - NT, :, wid],
                                  osem.at[tb]).wait()

        rows = rows_v.at[b]
        tile = tile_v.at[tb]
        pos16 = [pos_v[s, pl.ds(16 * dg, 16)] for dg in range(4)]

        @plsc.parallel_loop(0, BPW, unroll=8)
        def t_loop(t):
            colt = jnp.full((16,), t, jnp.int32)
            for dg in range(4):
                v = rows[t, pl.ds(16 * dg, 16)] + pos16[dg]
                plsc.store_scatter(tile, [db_idx[dg], di_idx[dg], colt], v)

        pltpu.async_copy(tile_v.at[tb, :, :, pl.ds(0, BPW)],
                         out_hbm.at[s, :, wid], osem.at[tb])

    for k in range(NT):
        s = SEQ - NT + k
        pltpu.make_async_copy(tile_v.at[s % NT, :, :, pl.ds(0, BPW)],
                              out_hbm.at[s, :, wid],
                              osem.at[s % NT]).wait()


@jax.jit
def kernel(x, token_emb, pos_emb):
    xt = x.T.astype(jnp.int32)
    pos = pos_emb[0, :SEQ, :]

    mesh = plsc.VectorSubcoreMesh(core_axis_name="c", subcore_axis_name="s")
    run = pl.kernel(
        _sc_kernel,
        out_type=jax.ShapeDtypeStruct((SEQ, DIM // 8, NW, 8, BPW), jnp.float32),
        mesh=mesh,
        scratch_types=[
            pltpu.VMEM((SEQ, DIM), jnp.float32),
            pltpu.VMEM((SEQ, BPW), jnp.int32),
            pltpu.VMEM((NB, BPW, DIM), jnp.float32),
            pltpu.VMEM((NT, DIM // 8, 8, 133), jnp.float32),
            pltpu.SemaphoreType.DMA((NB,)),
            pltpu.SemaphoreType.DMA((NT,)),
        ],
        compiler_params=pltpu.CompilerParams(
            use_tc_tiling_on_sc=False,
            needs_layout_passes=False,
            disable_bounds_checks=True,
        ),
    )
    out5 = run(xt, token_emb, pos)
    return jnp.transpose(out5, (2, 4, 0, 1, 3)).reshape(BATCH, SEQ, DIM)


# speedup vs baseline: 1.0466x; 1.0466x over previous
import jax
import jax.numpy as jnp
from jax import lax
from jax.experimental import pallas as pl
from jax.experimental.pallas import tpu as pltpu
from jax.experimental.pallas import tpu_sc as plsc

VOCAB = 1000000
DIM = 64
SEQ = 200
BATCH = 4096

NC, NS = 2, 16
NW = NC * NS
BPW = BATCH // NW
NB = 4
NT = 2


def _sc_kernel(xt_hbm, tab_hbm, pos_hbm, out_hbm, pos_v, idx_v, rows_v,
               tile_v, gsem, osem):
    wid = lax.axis_index("s") * NC + lax.axis_index("c")

    pltpu.sync_copy(pos_hbm, pos_v)
    pltpu.sync_copy(xt_hbm.at[:, pl.ds(wid * BPW, BPW)], idx_v)

    iota16 = lax.iota(jnp.int32, 16)
    db_idx = [iota16 // 8 + 2 * dg for dg in range(4)]
    di_idx = [lax.rem(iota16, 8)] * 4

    def gather(s, b):
        pltpu.async_copy(tab_hbm.at[idx_v.at[s]], rows_v.at[b], gsem.at[b])

    gather(0, 0)
    gather(1, 1)
    gather(2, 2)

    @pl.loop(0, SEQ)
    def s_loop(s):
        b = lax.rem(s, NB)
        tb = lax.rem(s, NT)

        @pl.when(s + 3 < SEQ)
        def _():
            gather(s + 3, lax.rem(s + 3, NB))

        pltpu.make_async_copy(tab_hbm.at[idx_v.at[s]], rows_v.at[b],
                              gsem.at[b]).wait()

        @pl.when(s >= NT)
        def _():
            pltpu.make_async_copy(tile_v.at[tb, :, :, pl.ds(0, BPW)],
                                  out_hbm.at[s - NT, :, wid],
                                  osem.at[tb]).wait()

        rows = rows_v.at[b]
        tile = tile_v.at[tb]
        pos16 = [pos_v[s, pl.ds(16 * dg, 16)] for dg in range(4)]

        @plsc.parallel_loop(0, BPW, unroll=4)
        def t_loop(t):
            colt = jnp.full((16,), t, jnp.int32)
            for dg in range(4):
                v = rows[t, pl.ds(16 * dg, 16)] + pos16[dg]
                plsc.store_scatter(tile, [db_idx[dg], di_idx[dg], colt], v)

        pltpu.async_copy(tile_v.at[tb, :, :, pl.ds(0, BPW)],
                         out_hbm.at[s, :, wid], osem.at[tb])

    for k in range(NT):
        s = SEQ - NT + k
        pltpu.make_async_copy(tile_v.at[s % NT, :, :, pl.ds(0, BPW)],
                              out_hbm.at[s, :, wid],
                              osem.at[s % NT]).wait()


@jax.jit
def kernel(x, token_emb, pos_emb):
    xt = x.T.astype(jnp.int32)
    pos = pos_emb[0, :SEQ, :]

    mesh = plsc.VectorSubcoreMesh(core_axis_name="c", subcore_axis_name="s")
    run = pl.kernel(
        _sc_kernel,
        out_type=jax.ShapeDtypeStruct((SEQ, DIM // 8, NW, 8, BPW), jnp.float32),
        mesh=mesh,
        scratch_types=[
            pltpu.VMEM((SEQ, DIM), jnp.float32),
            pltpu.VMEM((SEQ, BPW), jnp.int32),
            pltpu.VMEM((NB, BPW, DIM), jnp.float32),
            pltpu.VMEM((NT, DIM // 8, 8, 133), jnp.float32),
            pltpu.SemaphoreType.DMA((NB,)),
            pltpu.SemaphoreType.DMA((NT,)),
        ],
        compiler_params=pltpu.CompilerParams(
            use_tc_tiling_on_sc=False,
            needs_layout_passes=False,
            disable_bounds_checks=True,
        ),
    )
    out5 = run(xt, token_emb, pos)
    return jnp.transpose(out5, (2, 4, 0, 1, 3)).reshape(BATCH, SEQ, DIM)


# speedup vs baseline: 1.0497x; 1.0030x over previous
import jax
import jax.numpy as jnp
from jax import lax
from jax.experimental import pallas as pl
from jax.experimental.pallas import tpu as pltpu
from jax.experimental.pallas import tpu_sc as plsc

VOCAB = 1000000
DIM = 64
SEQ = 200
BATCH = 4096

NC, NS = 2, 16
NW = NC * NS
BPW = BATCH // NW
NB = 4
NT = 2


def _sc_kernel(xt_hbm, tab_hbm, pos_hbm, out_hbm, pos_v, idx_v, rows_v,
               tile_v, gsem, osem):
    wid = lax.axis_index("s") * NC + lax.axis_index("c")

    pltpu.sync_copy(pos_hbm, pos_v)
    pltpu.sync_copy(xt_hbm.at[wid], idx_v)

    iota16 = lax.iota(jnp.int32, 16)
    db_idx = [iota16 // 8 + 2 * dg for dg in range(4)]
    di_idx = [lax.rem(iota16, 8)] * 4

    def gather(s, b):
        pltpu.async_copy(tab_hbm.at[idx_v.at[s]], rows_v.at[b], gsem.at[b])

    gather(0, 0)
    gather(1, 1)
    gather(2, 2)

    @pl.loop(0, SEQ)
    def s_loop(s):
        b = lax.rem(s, NB)
        tb = lax.rem(s, NT)

        @pl.when(s + 3 < SEQ)
        def _():
            gather(s + 3, lax.rem(s + 3, NB))

        pltpu.make_async_copy(tab_hbm.at[idx_v.at[s]], rows_v.at[b],
                              gsem.at[b]).wait()

        @pl.when(s >= NT)
        def _():
            pltpu.make_async_copy(tile_v.at[tb, :, :, pl.ds(0, BPW)],
                                  out_hbm.at[s - NT, :, wid],
                                  osem.at[tb]).wait()

        rows = rows_v.at[b]
        tile = tile_v.at[tb]
        pos16 = [pos_v[s, pl.ds(16 * dg, 16)] for dg in range(4)]

        @plsc.parallel_loop(0, BPW, unroll=4)
        def t_loop(t):
            colt = jnp.full((16,), t, jnp.int32)
            for dg in range(4):
                v = rows[t, pl.ds(16 * dg, 16)] + pos16[dg]
                plsc.store_scatter(tile, [db_idx[dg], di_idx[dg], colt], v)

        pltpu.async_copy(tile_v.at[tb, :, :, pl.ds(0, BPW)],
                         out_hbm.at[s, :, wid], osem.at[tb])

    for k in range(NT):
        s = SEQ - NT + k
        pltpu.make_async_copy(tile_v.at[s % NT, :, :, pl.ds(0, BPW)],
                              out_hbm.at[s, :, wid],
                              osem.at[s % NT]).wait()


@jax.jit
def kernel(x, token_emb, pos_emb):
    xt = x.T.astype(jnp.int32).reshape(SEQ, NW, BPW).transpose(1, 0, 2)
    pos = pos_emb[0, :SEQ, :]

    mesh = plsc.VectorSubcoreMesh(core_axis_name="c", subcore_axis_name="s")
    run = pl.kernel(
        _sc_kernel,
        out_type=jax.ShapeDtypeStruct((SEQ, DIM // 8, NW, 8, BPW), jnp.float32),
        mesh=mesh,
        scratch_types=[
            pltpu.VMEM((SEQ, DIM), jnp.float32),
            pltpu.VMEM((SEQ, BPW), jnp.int32),
            pltpu.VMEM((NB, BPW, DIM), jnp.float32),
            pltpu.VMEM((NT, DIM // 8, 8, 133), jnp.float32),
            pltpu.SemaphoreType.DMA((NB,)),
            pltpu.SemaphoreType.DMA((NT,)),
        ],
        compiler_params=pltpu.CompilerParams(
            use_tc_tiling_on_sc=False,
            needs_layout_passes=False,
            disable_bounds_checks=True,
        ),
    )
    out5 = run(xt, token_emb, pos)
    return jnp.transpose(out5, (2, 4, 0, 1, 3)).reshape(BATCH, SEQ, DIM)
